# Initial kernel scaffold; baseline (speedup 1.0000x reference)
import jax
import jax.numpy as jnp
from jax.experimental import pallas as pl


def kernel(x, idx_mask, batch, Wp, W1, b1, W2, b2, W3, b3):
    raise NotImplementedError("write your pallas kernel here")


# speedup vs baseline: 3.6613x; 3.6613x over previous
import jax
import jax.numpy as jnp
from jax.experimental import pallas as pl
from jax.experimental.pallas import tpu as pltpu

N_NODES = 32768
D = 4096
G = 256
BLK = 512
NB = N_NODES // BLK


def _body(bb, mb, xb, wall, w1t, b1r, w2t, b2r, w3t, b3r, out, acc, cnt):
    i = pl.program_id(0)

    @pl.when(i == 0)
    def _init():
        acc[...] = jnp.zeros_like(acc)
        cnt[...] = jnp.zeros_like(cnt)

    b = bb[0]
    m = mb[0]
    seg_ids = jax.lax.broadcasted_iota(jnp.int32, (G, BLK), 0)
    oh = jnp.where(b == seg_ids, m, 0.0)
    acc[...] += jnp.dot(oh, xb[...], preferred_element_type=jnp.float32)
    cnt[...] += jnp.sum(oh, axis=1, keepdims=True)

    @pl.when(i == NB - 1)
    def _head():
        pooled = acc[...] / jnp.maximum(cnt[...], 1.0)
        no = jnp.dot(pooled, wall[...], preferred_element_type=jnp.float32)
        h = jnp.maximum(jnp.dot(no, w1t[...], preferred_element_type=jnp.float32) + b1r[...], 0.0)
        h = jnp.maximum(jnp.dot(h, w2t[...], preferred_element_type=jnp.float32) + b2r[...], 0.0)
        out[...] = jnp.dot(h, w3t[...], preferred_element_type=jnp.float32) + b3r[...]


def kernel(x, idx_mask, batch, Wp, W1, b1, W2, b2, W3, b3):
    batch_r = batch.reshape(NB, 1, BLK)
    mask_r = idx_mask.astype(jnp.float32).reshape(NB, 1, BLK)
    wall = Wp.transpose(0, 2, 1).reshape(D, 1024)
    w1t, w2t, w3t = W1.T, W2.T, W3.T
    b1r, b2r, b3r = b1.reshape(1, -1), b2.reshape(1, -1), b3.reshape(1, -1)

    return pl.pallas_call(
        _body,
        grid=(NB,),
        in_specs=[
            pl.BlockSpec((1, 1, BLK), lambda i: (i, 0, 0)),
            pl.BlockSpec((1, 1, BLK), lambda i: (i, 0, 0)),
            pl.BlockSpec((BLK, D), lambda i: (i, 0)),
            pl.BlockSpec((D, 1024), lambda i: (0, 0)),
            pl.BlockSpec((1024, 512), lambda i: (0, 0)),
            pl.BlockSpec((1, 512), lambda i: (0, 0)),
            pl.BlockSpec((512, 256), lambda i: (0, 0)),
            pl.BlockSpec((1, 256), lambda i: (0, 0)),
            pl.BlockSpec((256, 1195), lambda i: (0, 0)),
            pl.BlockSpec((1, 1195), lambda i: (0, 0)),
        ],
        out_specs=pl.BlockSpec((G, 1195), lambda i: (0, 0)),
        out_shape=jax.ShapeDtypeStruct((G, 1195), jnp.float32),
        scratch_shapes=[
            pltpu.VMEM((G, D), jnp.float32),
            pltpu.VMEM((G, 1), jnp.float32),
        ],
    )(batch_r, mask_r, x, wall, w1t, b1r, w2t, b2r, w3t, b3r)


# speedup vs baseline: 3.6733x; 1.0033x over previous
import jax
import jax.numpy as jnp
from jax.experimental import pallas as pl
from jax.experimental.pallas import tpu as pltpu

N_NODES = 32768
D = 4096
G = 256
BLK = 512
NB = N_NODES // BLK


def _body(bb, mb, xb, wall, w1t, b1r, w2t, b2r, w3t, b3r, out, acc, cnt):
    i = pl.program_id(0)

    @pl.when(i == 0)
    def _init():
        acc[...] = jnp.zeros_like(acc)
        cnt[...] = jnp.zeros_like(cnt)

    b = bb[0]
    m = mb[0]
    seg_ids = jax.lax.broadcasted_iota(jnp.int32, (G, BLK), 0)
    oh = jnp.where(b == seg_ids, m, 0.0)
    acc[...] += jnp.dot(oh.astype(jnp.bfloat16), xb[...].astype(jnp.bfloat16),
                        preferred_element_type=jnp.float32)
    cnt[...] += jnp.sum(oh, axis=1, keepdims=True)

    @pl.when(i == NB - 1)
    def _head():
        pooled = acc[...] / jnp.maximum(cnt[...], 1.0)
        no = jnp.dot(pooled, wall[...], preferred_element_type=jnp.float32)
        h = jnp.maximum(jnp.dot(no, w1t[...], preferred_element_type=jnp.float32) + b1r[...], 0.0)
        h = jnp.maximum(jnp.dot(h, w2t[...], preferred_element_type=jnp.float32) + b2r[...], 0.0)
        out[...] = jnp.dot(h, w3t[...], preferred_element_type=jnp.float32) + b3r[...]


def kernel(x, idx_mask, batch, Wp, W1, b1, W2, b2, W3, b3):
    batch_r = batch.reshape(NB, 1, BLK)
    mask_r = idx_mask.astype(jnp.float32).reshape(NB, 1, BLK)
    wall = Wp.transpose(0, 2, 1).reshape(D, 1024)
    w1t, w2t, w3t = W1.T, W2.T, W3.T
    b1r, b2r, b3r = b1.reshape(1, -1), b2.reshape(1, -1), b3.reshape(1, -1)

    return pl.pallas_call(
        _body,
        grid=(NB,),
        in_specs=[
            pl.BlockSpec((1, 1, BLK), lambda i: (i, 0, 0)),
            pl.BlockSpec((1, 1, BLK), lambda i: (i, 0, 0)),
            pl.BlockSpec((BLK, D), lambda i: (i, 0)),
            pl.BlockSpec((D, 1024), lambda i: (0, 0)),
            pl.BlockSpec((1024, 512), lambda i: (0, 0)),
            pl.BlockSpec((1, 512), lambda i: (0, 0)),
            pl.BlockSpec((512, 256), lambda i: (0, 0)),
            pl.BlockSpec((1, 256), lambda i: (0, 0)),
            pl.BlockSpec((256, 1195), lambda i: (0, 0)),
            pl.BlockSpec((1, 1195), lambda i: (0, 0)),
        ],
        out_specs=pl.BlockSpec((G, 1195), lambda i: (0, 0)),
        out_shape=jax.ShapeDtypeStruct((G, 1195), jnp.float32),
        scratch_shapes=[
            pltpu.VMEM((G, D), jnp.float32),
            pltpu.VMEM((G, 1), jnp.float32),
        ],
    )(batch_r, mask_r, x, wall, w1t, b1r, w2t, b2r, w3t, b3r)
